# baseline (speedup 1.0000x reference)
import functools

import jax
import jax.numpy as jnp
from jax import lax
from jax.experimental import pallas as pl
from jax.experimental.pallas import tpu as pltpu
from jax.experimental.pallas import tpu_sc as plsc

_NC = 2
_NS = 16
_NW = _NC * _NS
_LANES = 16

def _vreg_gather(v, c):
    return lax.gather(
        v, c[:, None],
        dimension_numbers=lax.GatherDimensionNumbers(
            offset_dims=(), collapsed_slice_dims=(0,), start_index_map=(0,)),
        slice_sizes=(1,),
        mode=lax.GatherScatterMode.PROMISE_IN_BOUNDS)


def _sc_gather_mul(F0, F1, idx_flat):
    B = idx_flat.shape[0] // 2
    K = F0.shape[1]
    b_per_w = B // _NW
    nchunk = b_per_w // _LANES
    rows_c = _LANES
    mesh = plsc.VectorSubcoreMesh(core_axis_name="c", subcore_axis_name="s")

    @functools.partial(
        pl.kernel,
        mesh=mesh,
        out_type=jax.ShapeDtypeStruct((B, K), jnp.float32),
        scratch_types=[
            pltpu.VMEM((2 * b_per_w,), jnp.int32),
            pltpu.VMEM((b_per_w,), jnp.int32),
            pltpu.VMEM((b_per_w,), jnp.int32),
            pltpu.VMEM((b_per_w, K), jnp.float32),
            pltpu.VMEM((b_per_w, K), jnp.float32),
            pltpu.SemaphoreType.DMA,
        ]
        + [pltpu.SemaphoreType.DMA] * (2 * nchunk)
        + [pltpu.SemaphoreType.DMA] * nchunk,
    )
    def sc_kernel(idx_hbm, f0_hbm, f1_hbm, out_hbm,
                  ia_v, i0_v, i1_v, r0_v, r1_v, sem_i, *sems):
        g_sems = sems[: 2 * nchunk]
        w_sems = sems[2 * nchunk:]
        wid = lax.axis_index("s") * _NC + lax.axis_index("c")
        base = wid * b_per_w
        pltpu.sync_copy(idx_hbm.at[pl.ds(2 * base, 2 * b_per_w)], ia_v)

        lane = lax.iota(jnp.int32, _LANES)
        c_ev = (lane * 2) % _LANES
        c_od = (lane * 2 + 1) % _LANES
        low = lane < (_LANES // 2)

        pending = []
        for c in range(nchunk):
            lo = c * rows_c
            v0 = ia_v[pl.ds(2 * lo, _LANES)]
            v1 = ia_v[pl.ds(2 * lo + _LANES, _LANES)]
            i0_v[pl.ds(lo, rows_c)] = jnp.where(
                low, _vreg_gather(v0, c_ev), _vreg_gather(v1, c_ev))
            i1_v[pl.ds(lo, rows_c)] = jnp.where(
                low, _vreg_gather(v0, c_od), _vreg_gather(v1, c_od))
            g0 = pltpu.async_copy(f0_hbm.at[i0_v.at[pl.ds(lo, rows_c)]],
                                  r0_v.at[pl.ds(lo, rows_c)], g_sems[2 * c])
            g1 = pltpu.async_copy(f1_hbm.at[i1_v.at[pl.ds(lo, rows_c)]],
                                  r1_v.at[pl.ds(lo, rows_c)], g_sems[2 * c + 1])
            pending.append((g0, g1))

        writes = []
        for c in range(nchunk):
            lo = c * rows_c
            pending[c][0].wait()
            pending[c][1].wait()

            def row_body(r, carry):
                for j in range(K // _LANES):
                    sl = pl.ds(j * _LANES, _LANES)
                    r0_v[r, sl] = r0_v[r, sl] * r1_v[r, sl]
                return carry

            lax.fori_loop(lo, lo + rows_c, row_body, 0, unroll=2)
            writes.append(pltpu.async_copy(
                r0_v.at[pl.ds(lo, rows_c)],
                out_hbm.at[pl.ds(base + lo, rows_c)], w_sems[c]))
        for w in writes:
            w.wait()

    return sc_kernel(idx_flat, F0, F1)


def _tc_matmul_t_chunk(prod_c, F2, acc, chunk, n_chunks):
    Bc, K = prod_c.shape
    N = F2.shape[0]
    B = Bc * n_chunks
    BLK = 1024
    nblk = Bc // BLK
    off = chunk * nblk

    def mm_body(f2_ref, p_ref, *rest):
        o_ref = rest[-1]
        o_ref[...] = lax.dot_general(
            f2_ref[...], p_ref[...],
            (((1,), (1,)), ((), ())),
            preferred_element_type=jnp.float32,
        )

    in_specs = [
        pl.BlockSpec((N, K), lambda i: (0, 0)),
        pl.BlockSpec((BLK, K), lambda i: (i, 0)),
    ]
    operands = [F2, prod_c]
    kwargs = {}
    if acc is not None:
        in_specs.append(pl.BlockSpec(memory_space=pl.ANY))
        operands.append(acc)
        kwargs["input_output_aliases"] = {2: 0}

    return pl.pallas_call(
        mm_body,
        grid=(nblk,),
        in_specs=in_specs,
        out_specs=pl.BlockSpec((N, BLK), lambda i: (0, i + off)),
        out_shape=jax.ShapeDtypeStruct((N, B), jnp.float32),
        **kwargs,
    )(*operands)


_TOP_CHUNKS = 2


def kernel(indices, F0, F1, F2, log_sigma):
    idx_flat = indices.astype(jnp.int32).reshape(-1)
    B = indices.shape[0]
    Bc = B // _TOP_CHUNKS
    prods = [
        _sc_gather_mul(F0, F1,
                       lax.dynamic_slice_in_dim(idx_flat, 2 * c * Bc, 2 * Bc))
        for c in range(_TOP_CHUNKS)
    ]
    res_t = None
    for c, p in enumerate(prods):
        res_t = _tc_matmul_t_chunk(p, F2, res_t, c, _TOP_CHUNKS)
    return (res_t.T, log_sigma)


# speedup vs baseline: 1.0818x; 1.0818x over previous
import functools

import jax
import jax.numpy as jnp
from jax import lax
from jax.experimental import pallas as pl
from jax.experimental.pallas import tpu as pltpu
from jax.experimental.pallas import tpu_sc as plsc

_NC = 2
_NS = 16
_NW = _NC * _NS
_LANES = 16

def _vreg_gather(v, c):
    return lax.gather(
        v, c[:, None],
        dimension_numbers=lax.GatherDimensionNumbers(
            offset_dims=(), collapsed_slice_dims=(0,), start_index_map=(0,)),
        slice_sizes=(1,),
        mode=lax.GatherScatterMode.PROMISE_IN_BOUNDS)


def _sc_gather_mul(F0, F1, idx_flat):
    B = idx_flat.shape[0] // 2
    K = F0.shape[1]
    b_per_w = B // _NW
    nchunk = b_per_w // _LANES
    rows_c = _LANES
    mesh = plsc.VectorSubcoreMesh(core_axis_name="c", subcore_axis_name="s")

    @functools.partial(
        pl.kernel,
        mesh=mesh,
        out_type=jax.ShapeDtypeStruct((B, K), jnp.float32),
        scratch_types=[
            pltpu.VMEM((2 * b_per_w,), jnp.int32),
            pltpu.VMEM((b_per_w,), jnp.int32),
            pltpu.VMEM((b_per_w,), jnp.int32),
            pltpu.VMEM((b_per_w, K), jnp.float32),
            pltpu.VMEM((b_per_w, K), jnp.float32),
            pltpu.SemaphoreType.DMA,
        ]
        + [pltpu.SemaphoreType.DMA] * (2 * nchunk)
        + [pltpu.SemaphoreType.DMA] * nchunk,
    )
    def sc_kernel(idx_hbm, f0_hbm, f1_hbm, out_hbm,
                  ia_v, i0_v, i1_v, r0_v, r1_v, sem_i, *sems):
        g_sems = sems[: 2 * nchunk]
        w_sems = sems[2 * nchunk:]
        wid = lax.axis_index("s") * _NC + lax.axis_index("c")
        base = wid * b_per_w
        pltpu.sync_copy(idx_hbm.at[pl.ds(2 * base, 2 * b_per_w)], ia_v)

        lane = lax.iota(jnp.int32, _LANES)
        c_ev = (lane * 2) % _LANES
        c_od = (lane * 2 + 1) % _LANES
        low = lane < (_LANES // 2)

        pending = []
        for c in range(nchunk):
            lo = c * rows_c
            v0 = ia_v[pl.ds(2 * lo, _LANES)]
            v1 = ia_v[pl.ds(2 * lo + _LANES, _LANES)]
            i0_v[pl.ds(lo, rows_c)] = jnp.where(
                low, _vreg_gather(v0, c_ev), _vreg_gather(v1, c_ev))
            i1_v[pl.ds(lo, rows_c)] = jnp.where(
                low, _vreg_gather(v0, c_od), _vreg_gather(v1, c_od))
            g0 = pltpu.async_copy(f0_hbm.at[i0_v.at[pl.ds(lo, rows_c)]],
                                  r0_v.at[pl.ds(lo, rows_c)], g_sems[2 * c])
            g1 = pltpu.async_copy(f1_hbm.at[i1_v.at[pl.ds(lo, rows_c)]],
                                  r1_v.at[pl.ds(lo, rows_c)], g_sems[2 * c + 1])
            pending.append((g0, g1))

        writes = []
        for c in range(nchunk):
            lo = c * rows_c
            pending[c][0].wait()
            pending[c][1].wait()

            def row_body(r, carry):
                for j in range(K // _LANES):
                    sl = pl.ds(j * _LANES, _LANES)
                    r0_v[r, sl] = r0_v[r, sl] * r1_v[r, sl]
                return carry

            lax.fori_loop(lo, lo + rows_c, row_body, 0, unroll=2)
            writes.append(pltpu.async_copy(
                r0_v.at[pl.ds(lo, rows_c)],
                out_hbm.at[pl.ds(base + lo, rows_c)], w_sems[c]))
        for w in writes:
            w.wait()

    return sc_kernel(idx_flat, F0, F1)


def _tc_matmul_t_chunk(prod_c, F2, acc, chunk, n_chunks):
    Bc, K = prod_c.shape
    N = F2.shape[0]
    B = Bc * n_chunks
    BLK = 1024
    nblk = Bc // BLK
    off = chunk * nblk

    def mm_body(f2_ref, p_ref, *rest):
        o_ref = rest[-1]
        o_ref[...] = lax.dot_general(
            f2_ref[...], p_ref[...],
            (((1,), (1,)), ((), ())),
            preferred_element_type=jnp.float32,
        )

    in_specs = [
        pl.BlockSpec((N, K), lambda i: (0, 0)),
        pl.BlockSpec((BLK, K), lambda i: (i, 0)),
    ]
    operands = [F2, prod_c]
    kwargs = {}
    if acc is not None:
        in_specs.append(pl.BlockSpec(memory_space=pl.ANY))
        operands.append(acc)
        kwargs["input_output_aliases"] = {2: 0}

    return pl.pallas_call(
        mm_body,
        grid=(nblk,),
        in_specs=in_specs,
        out_specs=pl.BlockSpec((N, BLK), lambda i: (0, i + off)),
        out_shape=jax.ShapeDtypeStruct((N, B), jnp.float32),
        **kwargs,
    )(*operands)


def kernel(indices, F0, F1, F2, log_sigma):
    idx_flat = indices.astype(jnp.int32).reshape(-1)
    prod = _sc_gather_mul(F0, F1, idx_flat)
    res_t = _tc_matmul_t_chunk(prod, F2, None, 0, 1)
    return (res_t.T, log_sigma)
